# baseline (speedup 1.0000x reference)
import functools
import math

import jax
import jax.numpy as jnp
from jax import lax
from jax.experimental import pallas as pl
from jax.experimental.pallas import tpu as pltpu
from jax.experimental.pallas import tpu_sc as plsc

D_MODEL = 64
SCALE = math.sqrt(D_MODEL)

NUM_CORES = 2
NUM_SUBCORES = 16
NUM_LANES = 16
NW = NUM_CORES * NUM_SUBCORES

CHUNK = 128


def _emb_kernel(n_chunks, idx_hbm, table_hbm, out_hbm, idx_v, rows_v, gsem):
    wid = lax.axis_index("s") * NUM_CORES + lax.axis_index("c")
    chunk_base = wid * n_chunks

    pltpu.sync_copy(idx_hbm.at[pl.ds(chunk_base, n_chunks)], idx_v)

    def body(g, carry):
        pltpu.async_copy(table_hbm.at[idx_v.at[g]], rows_v, gsem).wait()

        def scale(k, c):
            i = k // (D_MODEL // NUM_LANES)
            j = k % (D_MODEL // NUM_LANES)
            rows_v[i, pl.ds(j * NUM_LANES, NUM_LANES)] = (
                rows_v[i, pl.ds(j * NUM_LANES, NUM_LANES)] * SCALE
            )
            return c
        lax.fori_loop(0, CHUNK * D_MODEL // NUM_LANES, scale, 0)

        pltpu.sync_copy(
            rows_v, out_hbm.at[pl.ds((chunk_base + g) * CHUNK, CHUNK)]
        )
        return carry

    lax.fori_loop(0, n_chunks, body, 0)


def kernel(x, lut):
    orig_shape = x.shape
    b = x.size
    assert b % (NW * CHUNK) == 0
    n_chunks = b // (NW * CHUNK)

    idx2 = x.reshape(-1, CHUNK)

    mesh = plsc.VectorSubcoreMesh(
        core_axis_name="c", subcore_axis_name="s"
    )
    run = pl.kernel(
        functools.partial(_emb_kernel, n_chunks),
        out_type=jax.ShapeDtypeStruct((b, D_MODEL), jnp.float32),
        mesh=mesh,
        scratch_types=[
            pltpu.VMEM((n_chunks, CHUNK), jnp.int32),
            pltpu.VMEM((CHUNK, D_MODEL), jnp.float32),
            pltpu.SemaphoreType.DMA,
        ],
        compiler_params=pltpu.CompilerParams(use_tc_tiling_on_sc=False),
    )
    out = run(idx2, lut)
    return out.reshape(*orig_shape, D_MODEL)


# speedup vs baseline: 1.5320x; 1.5320x over previous
import functools
import math

import jax
import jax.numpy as jnp
from jax import lax
from jax.experimental import pallas as pl
from jax.experimental.pallas import tpu as pltpu
from jax.experimental.pallas import tpu_sc as plsc

D_MODEL = 64
SCALE = math.sqrt(D_MODEL)

NUM_CORES = 2
NUM_SUBCORES = 16
NUM_LANES = 16
NW = NUM_CORES * NUM_SUBCORES

CHUNK = 128
NBUF = 4
VECS = D_MODEL // NUM_LANES


def _emb_kernel(n_chunks, idx_hbm, table_hbm, out_hbm, idx_v, rows, gsems, ssems):
    wid = lax.axis_index("s") * NUM_CORES + lax.axis_index("c")
    chunk_base = wid * n_chunks

    pltpu.sync_copy(idx_hbm.at[pl.ds(chunk_base, n_chunks)], idx_v)

    def gather(c, b):
        return pltpu.make_async_copy(table_hbm.at[idx_v.at[c]], rows[b], gsems[b])

    def scatter(c, b):
        return pltpu.make_async_copy(
            rows[b], out_hbm.at[pl.ds((chunk_base + c) * CHUNK, CHUNK)], ssems[b]
        )

    gather(0, 0).start()

    def outer(g0):
        for b in range(NBUF):
            c = g0 * NBUF + b
            nb = (b + 1) % NBUF

            @pl.when(c + 1 < n_chunks)
            def _():
                @pl.when(c + 1 >= NBUF)
                def _():
                    scatter(c + 1 - NBUF, nb).wait()
                gather(c + 1, nb).start()

            gather(c, b).wait()

            def scale(i, carry):
                for j in range(VECS):
                    sl = pl.ds(j * NUM_LANES, NUM_LANES)
                    rows[b][i, sl] = rows[b][i, sl] * SCALE
                return carry
            lax.fori_loop(0, CHUNK, scale, 0)

            scatter(c, b).start()

    pl.loop(0, n_chunks // NBUF)(outer)

    for b in range(NBUF):
        scatter(n_chunks - NBUF + b, b).wait()


def kernel(x, lut):
    orig_shape = x.shape
    b = x.size
    assert b % (NW * CHUNK) == 0
    n_chunks = b // (NW * CHUNK)
    assert n_chunks % NBUF == 0

    idx2 = x.reshape(-1, CHUNK)

    mesh = plsc.VectorSubcoreMesh(core_axis_name="c", subcore_axis_name="s")
    run = pl.kernel(
        functools.partial(_emb_kernel, n_chunks),
        out_type=jax.ShapeDtypeStruct((b, D_MODEL), jnp.float32),
        mesh=mesh,
        scratch_types=[
            pltpu.VMEM((n_chunks, CHUNK), jnp.int32),
            [pltpu.VMEM((CHUNK, D_MODEL), jnp.float32) for _ in range(NBUF)],
            [pltpu.SemaphoreType.DMA for _ in range(NBUF)],
            [pltpu.SemaphoreType.DMA for _ in range(NBUF)],
        ],
        compiler_params=pltpu.CompilerParams(use_tc_tiling_on_sc=False),
    )
    out = run(idx2, lut)
    return out.reshape(*orig_shape, D_MODEL)
